# Initial kernel scaffold; baseline (speedup 1.0000x reference)
import jax
import jax.numpy as jnp
from jax.experimental import pallas as pl


def kernel(x, edge_attr, atom_table, bond_table):
    raise NotImplementedError("write your pallas kernel here")


# speedup vs baseline: 1.7855x; 1.7855x over previous
import functools

import jax
import jax.numpy as jnp
from jax import lax
from jax.experimental import pallas as pl
from jax.experimental.pallas import tpu as pltpu
from jax.experimental.pallas import tpu_sc as plsc

D = 128
CH = 400
NC, NS = 2, 16
NW = NC * NS

N_NODES_, NAF_ = 10000, 9
N_EDGES_, NBF_ = 320000, 3
ATOM_B = N_NODES_ * NAF_
BOND_B = N_EDGES_ * NBF_

ATOM_PER_W = 2800
ATOM_CHUNKS = ATOM_PER_W // CH
ATOM_TAIL = NW * ATOM_PER_W
BOND_PER_W = BOND_B // NW
BOND_CHUNKS = BOND_PER_W // CH


def _sc_gather(x_flat, e_flat, atom_table, bond_table):
  mesh = plsc.VectorSubcoreMesh(core_axis_name="c", subcore_axis_name="s")

  @functools.partial(
      pl.kernel,
      out_type=(
          jax.ShapeDtypeStruct((ATOM_B, D), jnp.float32),
          jax.ShapeDtypeStruct((BOND_B, D), jnp.float32),
      ),
      mesh=mesh,
      scratch_types=[
          pltpu.VMEM((CH,), jnp.int32),
          pltpu.VMEM((CH, D), jnp.float32),
          pltpu.SemaphoreType.DMA,
      ],
  )
  def k(x_hbm, e_hbm, at_hbm, bt_hbm, aout_hbm, bout_hbm, idx_v, rows_v, sem):
    wid = lax.axis_index("s") * NC + lax.axis_index("c")

    def do_chunk(idx_hbm, tab_hbm, out_hbm, base):
      pltpu.sync_copy(idx_hbm.at[pl.ds(base, CH)], idx_v)
      pltpu.async_copy(tab_hbm.at[idx_v], rows_v, sem).wait()
      pltpu.sync_copy(rows_v, out_hbm.at[pl.ds(base, CH)])

    abase = wid * ATOM_PER_W

    def atom_body(i, carry):
      do_chunk(x_hbm, at_hbm, aout_hbm, abase + i * CH)
      return carry

    lax.fori_loop(0, ATOM_CHUNKS, atom_body, 0)

    @pl.when(wid == 0)
    def _():
      do_chunk(x_hbm, at_hbm, aout_hbm, ATOM_TAIL)

    bbase = wid * BOND_PER_W

    def bond_body(i, carry):
      do_chunk(e_hbm, bt_hbm, bout_hbm, bbase + i * CH)
      return carry

    lax.fori_loop(0, BOND_CHUNKS, bond_body, 0)

  return k(x_flat, e_flat, atom_table, bond_table)


def kernel(x, edge_attr, atom_table, bond_table):
  x_flat = x.reshape(ATOM_B)
  e_flat = edge_attr.reshape(BOND_B)
  a_out, b_out = _sc_gather(x_flat, e_flat, atom_table, bond_table)
  return (a_out.reshape(N_NODES_, NAF_, D), b_out.reshape(N_EDGES_, NBF_, D))


# speedup vs baseline: 2.3316x; 1.3058x over previous
import functools

import jax
import jax.numpy as jnp
from jax import lax
from jax.experimental import pallas as pl
from jax.experimental.pallas import tpu as pltpu
from jax.experimental.pallas import tpu_sc as plsc

D = 128
CH = 400
NC, NS = 2, 16
NW = NC * NS

N_NODES_, NAF_ = 10000, 9
N_EDGES_, NBF_ = 320000, 3
ATOM_B = N_NODES_ * NAF_
BOND_B = N_EDGES_ * NBF_
ATOM_V, BOND_V = 1152, 384

ATOM_PER_W = 2800
ATOM_CHUNKS = ATOM_PER_W // CH
ATOM_TAIL = NW * ATOM_PER_W
BOND_PER_W = BOND_B // NW
BOND_CHUNKS = BOND_PER_W // CH


def _sc_gather(x_flat, e_flat, atom_table, bond_table):
  mesh = plsc.VectorSubcoreMesh(core_axis_name="c", subcore_axis_name="s")

  @functools.partial(
      pl.kernel,
      out_type=(
          jax.ShapeDtypeStruct((ATOM_B, D), jnp.float32),
          jax.ShapeDtypeStruct((BOND_B, D), jnp.float32),
      ),
      mesh=mesh,
      scratch_types=[
          pltpu.VMEM_SHARED((ATOM_V, D), jnp.float32),
          pltpu.VMEM_SHARED((BOND_V, D), jnp.float32),
          pltpu.VMEM((CH,), jnp.int32),
          pltpu.VMEM((CH,), jnp.int32),
          pltpu.VMEM((CH, D), jnp.float32),
          pltpu.VMEM((CH, D), jnp.float32),
          pltpu.SemaphoreType.DMA,
          pltpu.SemaphoreType.DMA,
          pltpu.SemaphoreType.DMA,
          pltpu.SemaphoreType.DMA,
      ],
  )
  def k(x_hbm, e_hbm, at_hbm, bt_hbm, aout_hbm, bout_hbm,
        at_sp, bt_sp, iA, iB, rA, rB, gsA, gsB, ssA, ssB):
    sid = lax.axis_index("s")
    wid = sid * NC + lax.axis_index("c")

    @pl.when(sid == 0)
    def _():
      pltpu.sync_copy(at_hbm, at_sp)
      pltpu.sync_copy(bt_hbm, bt_sp)

    plsc.subcore_barrier()

    def pipe(idx_hbm, tab_sp, out_hbm, base, nch):
      def start_gather(ibuf, rbuf, sem, c):
        pltpu.sync_copy(idx_hbm.at[pl.ds(base + c * CH, CH)], ibuf)
        return pltpu.async_copy(tab_sp.at[ibuf], rbuf, sem)

      def start_scatter(rbuf, sem, c):
        return pltpu.async_copy(rbuf, out_hbm.at[pl.ds(base + c * CH, CH)], sem)

      def drain_scatter(rbuf, sem, c):
        pltpu.make_async_copy(rbuf, out_hbm.at[pl.ds(base + c * CH, CH)], sem).wait()

      def pair(t, carry):
        c0 = 2 * t
        c1 = c0 + 1
        @pl.when(t > 0)
        def _():
          drain_scatter(rA, ssA, c0)
          drain_scatter(rB, ssB, c1)

        gA = start_gather(iA, rA, gsA, c0)
        gB = start_gather(iB, rB, gsB, c1)
        gA.wait()
        start_scatter(rA, ssA, c0)
        gB.wait()
        start_scatter(rB, ssB, c1)
        return carry

      lax.fori_loop(0, nch // 2, pair, 0)
      drain_scatter(rA, ssA, 0)
      drain_scatter(rB, ssB, 0)
      if nch % 2:
        c = nch - 1
        g = start_gather(iA, rA, gsA, c)
        g.wait()
        start_scatter(rA, ssA, c).wait()

    pipe(x_hbm, at_sp, aout_hbm, wid * ATOM_PER_W, ATOM_CHUNKS)

    @pl.when(wid == 0)
    def _():
      pltpu.sync_copy(x_hbm.at[pl.ds(ATOM_TAIL, CH)], iA)
      pltpu.async_copy(at_sp.at[iA], rA, gsA).wait()
      pltpu.async_copy(rA, aout_hbm.at[pl.ds(ATOM_TAIL, CH)], ssA).wait()

    pipe(e_hbm, bt_sp, bout_hbm, wid * BOND_PER_W, BOND_CHUNKS)

  return k(x_flat, e_flat, atom_table, bond_table)


def kernel(x, edge_attr, atom_table, bond_table):
  x_flat = x.reshape(ATOM_B)
  e_flat = edge_attr.reshape(BOND_B)
  a_out, b_out = _sc_gather(x_flat, e_flat, atom_table, bond_table)
  return (a_out.reshape(N_NODES_, NAF_, D), b_out.reshape(N_EDGES_, NBF_, D))


# speedup vs baseline: 4.1780x; 1.7919x over previous
import functools

import jax
import jax.numpy as jnp
from jax import lax
from jax.experimental import pallas as pl
from jax.experimental.pallas import tpu as pltpu
from jax.experimental.pallas import tpu_sc as plsc

D = 128
NC, NS = 2, 16
NW = NC * NS

N_NODES_, NAF_ = 10000, 9
N_EDGES_, NBF_ = 320000, 3
ATOM_V, BOND_V = 1152, 384

CHN = 8
ATOM_CHUNKS_W = 39
ATOM_PER_W = ATOM_CHUNKS_W * CHN
ATOM_TAIL0 = NW * ATOM_PER_W
ATOM_TAIL1 = ATOM_TAIL0 + CHN

CHE = 80
BOND_PER_W = N_EDGES_ // NW
BOND_CHUNKS_W = BOND_PER_W // CHE


def _sc_gather(x_flat, e_flat, atom_table, bond_table):
  mesh = plsc.VectorSubcoreMesh(core_axis_name="c", subcore_axis_name="s")

  @functools.partial(
      pl.kernel,
      out_type=(
          jax.ShapeDtypeStruct((N_NODES_, NAF_, D), jnp.float32),
          jax.ShapeDtypeStruct((N_EDGES_, NBF_, D), jnp.float32),
      ),
      mesh=mesh,
      scratch_types=[
          pltpu.VMEM_SHARED((ATOM_V, D), jnp.float32),
          pltpu.VMEM_SHARED((BOND_V, D), jnp.float32),
          pltpu.VMEM((CHN * NAF_,), jnp.int32),
          pltpu.VMEM((CHN * NAF_,), jnp.int32),
          pltpu.VMEM((CHE * NBF_,), jnp.int32),
          pltpu.VMEM((CHE * NBF_,), jnp.int32),
          pltpu.VMEM((CHN * NAF_, D), jnp.float32),
          pltpu.VMEM((CHN * NAF_, D), jnp.float32),
          pltpu.VMEM((CHE * NBF_, D), jnp.float32),
          pltpu.VMEM((CHE * NBF_, D), jnp.float32),
          pltpu.SemaphoreType.DMA,
          pltpu.SemaphoreType.DMA,
          pltpu.SemaphoreType.DMA,
          pltpu.SemaphoreType.DMA,
      ],
  )
  def k(x_hbm, e_hbm, at_hbm, bt_hbm, aout_hbm, bout_hbm,
        at_sp, bt_sp, aiA, aiB, biA, biB, arA, arB, brA, brB,
        gsA, gsB, ssA, ssB):
    sid = lax.axis_index("s")
    wid = sid * NC + lax.axis_index("c")

    @pl.when(sid == 0)
    def _():
      pltpu.sync_copy(at_hbm, at_sp)
      pltpu.sync_copy(bt_hbm, bt_sp)

    plsc.subcore_barrier()

    def make_phase(idx_hbm, tab_sp, out_hbm, slab, feat, base, iA, iB, rA, rB):
      def start_gather(ibuf, rbuf, sem, c):
        r0 = (base + c * slab) * feat
        pltpu.sync_copy(idx_hbm.at[pl.ds(r0, slab * feat)], ibuf)
        return pltpu.async_copy(tab_sp.at[ibuf], rbuf, sem)

      def start_scatter(rbuf, sem, c):
        s0 = base + c * slab
        return pltpu.async_copy(
            rbuf.reshape(slab, feat, D), out_hbm.at[pl.ds(s0, slab)], sem)

      def drain_scatter(rbuf, sem, c):
        s0 = base + c * slab
        pltpu.make_async_copy(
            rbuf.reshape(slab, feat, D), out_hbm.at[pl.ds(s0, slab)], sem).wait()

      def pipe(nch):
        def pair(t, carry):
          c0 = 2 * t
          c1 = c0 + 1
          @pl.when(t > 0)
          def _():
            drain_scatter(rA, ssA, c0)
            drain_scatter(rB, ssB, c1)

          gA = start_gather(iA, rA, gsA, c0)
          gB = start_gather(iB, rB, gsB, c1)
          gA.wait()
          start_scatter(rA, ssA, c0)
          gB.wait()
          start_scatter(rB, ssB, c1)
          return carry

        lax.fori_loop(0, nch // 2, pair, 0)
        drain_scatter(rA, ssA, 0)
        drain_scatter(rB, ssB, 0)
        if nch % 2:
          c = nch - 1
          g = start_gather(iA, rA, gsA, c)
          g.wait()
          start_scatter(rA, ssA, c)
          drain_scatter(rA, ssA, c)

      return start_gather, start_scatter, drain_scatter, pipe

    a_ops = make_phase(x_hbm, at_sp, aout_hbm, CHN, NAF_, wid * ATOM_PER_W,
                       aiA, aiB, arA, arB)
    a_ops[3](ATOM_CHUNKS_W)

    for w, tail_base in ((0, ATOM_TAIL0), (1, ATOM_TAIL1)):
      @pl.when(wid == w)
      def _(tail_base=tail_base):
        sg, ss, dsc, _ = make_phase(x_hbm, at_sp, aout_hbm, CHN, NAF_,
                                    tail_base, aiA, aiB, arA, arB)
        g = sg(aiA, arA, gsA, 0)
        g.wait()
        ss(arA, ssA, 0)
        dsc(arA, ssA, 0)

    b_ops = make_phase(e_hbm, bt_sp, bout_hbm, CHE, NBF_, wid * BOND_PER_W,
                       biA, biB, brA, brB)
    b_ops[3](BOND_CHUNKS_W)

  return k(x_flat, e_flat, atom_table, bond_table)


def kernel(x, edge_attr, atom_table, bond_table):
  x_flat = x.reshape(N_NODES_ * NAF_)
  e_flat = edge_attr.reshape(N_EDGES_ * NBF_)
  return _sc_gather(x_flat, e_flat, atom_table, bond_table)


# speedup vs baseline: 10.7228x; 2.5665x over previous
import functools

import jax
import jax.numpy as jnp
from jax import lax
from jax.experimental import pallas as pl
from jax.experimental.pallas import tpu as pltpu
from jax.experimental.pallas import tpu_sc as plsc

D = 128
NC, NS = 2, 16
NW = NC * NS

N_NODES_, NAF_ = 10000, 9
N_EDGES_, NBF_ = 320000, 3
ATOM_V, BOND_V = 1152, 384
ATOM_B = N_NODES_ * NAF_
BOND_B = N_EDGES_ * NBF_

CHA = 104
ATOM_NCH = 3
ATOM_TAIL = NW * ATOM_NCH * CHA
CHB = 200
BOND_NCH = 50


def _sc_gather(x_flat, e_flat, atom_table, bond_table):
  mesh = plsc.VectorSubcoreMesh(core_axis_name="c", subcore_axis_name="s")

  @functools.partial(
      pl.kernel,
      out_type=(
          jax.ShapeDtypeStruct((ATOM_B, D), jnp.float32),
          jax.ShapeDtypeStruct((BOND_B, D), jnp.float32),
      ),
      mesh=mesh,
      scratch_types=[
          pltpu.VMEM_SHARED((ATOM_V, D), jnp.float32),
          pltpu.VMEM_SHARED((BOND_V, D), jnp.float32),
          pltpu.VMEM((CHA,), jnp.int32),
          pltpu.VMEM((CHA,), jnp.int32),
          pltpu.VMEM((CHB,), jnp.int32),
          pltpu.VMEM((CHB,), jnp.int32),
          pltpu.VMEM((16,), jnp.int32),
          pltpu.VMEM((CHA, D), jnp.float32),
          pltpu.VMEM((CHA, D), jnp.float32),
          pltpu.VMEM((CHB, D), jnp.float32),
          pltpu.VMEM((CHB, D), jnp.float32),
          pltpu.VMEM((16, D), jnp.float32),
          pltpu.SemaphoreType.DMA,
          pltpu.SemaphoreType.DMA,
          pltpu.SemaphoreType.DMA,
          pltpu.SemaphoreType.DMA,
      ],
  )
  def k(x_hbm, e_hbm, at_hbm, bt_hbm, aout_hbm, bout_hbm,
        at_sp, bt_sp, aiA, aiB, biA, biB, ti, arA, arB, brA, brB, tr,
        gsA, gsB, ssA, ssB):
    sid = lax.axis_index("s")
    wid = sid * NC + lax.axis_index("c")

    @pl.when(sid == 0)
    def _():
      pltpu.sync_copy(at_hbm, at_sp)
      pltpu.sync_copy(bt_hbm, bt_sp)

    plsc.subcore_barrier()

    def pipe(idx_hbm, tab_sp, out_hbm, base, ch, nch, iA, iB, rA, rB):
      def start_gather(ibuf, rbuf, sem, c):
        pltpu.sync_copy(idx_hbm.at[pl.ds(base + c * ch, ch)], ibuf)
        return pltpu.async_copy(tab_sp.at[ibuf], rbuf, sem)

      def start_scatter(rbuf, sem, c):
        return pltpu.async_copy(rbuf, out_hbm.at[pl.ds(base + c * ch, ch)], sem)

      def drain_scatter(rbuf, sem, c):
        pltpu.make_async_copy(
            rbuf, out_hbm.at[pl.ds(base + c * ch, ch)], sem).wait()

      def pair(t, carry):
        c0 = 2 * t
        c1 = c0 + 1
        @pl.when(t > 0)
        def _():
          drain_scatter(rA, ssA, c0)
          drain_scatter(rB, ssB, c1)

        gA = start_gather(iA, rA, gsA, c0)
        gB = start_gather(iB, rB, gsB, c1)
        gA.wait()
        start_scatter(rA, ssA, c0)
        gB.wait()
        start_scatter(rB, ssB, c1)
        return carry

      lax.fori_loop(0, nch // 2, pair, 0)
      drain_scatter(rA, ssA, 0)
      drain_scatter(rB, ssB, 0)
      if nch % 2:
        c = nch - 1
        g = start_gather(iA, rA, gsA, c)
        g.wait()
        start_scatter(rA, ssA, c)
        drain_scatter(rA, ssA, c)

    for f in range(NAF_):
      pipe(x_hbm, at_sp, aout_hbm, f * N_NODES_ + wid * (ATOM_NCH * CHA),
           CHA, ATOM_NCH, aiA, aiB, arA, arB)

    for f in range(NAF_):
      @pl.when(wid == f)
      def _(f=f):
        base = f * N_NODES_ + ATOM_TAIL
        pltpu.sync_copy(x_hbm.at[pl.ds(base, 16)], ti)
        pltpu.async_copy(at_sp.at[ti], tr, gsA).wait()
        pltpu.async_copy(tr, aout_hbm.at[pl.ds(base, 16)], ssA).wait()

    for f in range(NBF_):
      pipe(e_hbm, bt_sp, bout_hbm, f * N_EDGES_ + wid * (BOND_NCH * CHB),
           CHB, BOND_NCH, biA, biB, brA, brB)

  return k(x_flat, e_flat, atom_table, bond_table)


def kernel(x, edge_attr, atom_table, bond_table):
  x_flat = x.T.reshape(ATOM_B)
  e_flat = edge_attr.T.reshape(BOND_B)
  a_out, b_out = _sc_gather(x_flat, e_flat, atom_table, bond_table)
  return (
      a_out.reshape(NAF_, N_NODES_, D).transpose(1, 0, 2),
      b_out.reshape(NBF_, N_EDGES_, D).transpose(1, 0, 2),
  )


# speedup vs baseline: 11.5459x; 1.0768x over previous
import functools

import jax
import jax.numpy as jnp
from jax import lax
from jax.experimental import pallas as pl
from jax.experimental.pallas import tpu as pltpu
from jax.experimental.pallas import tpu_sc as plsc

D = 128
NC, NS = 2, 16
NW = NC * NS

N_NODES_, NAF_ = 10000, 9
N_EDGES_, NBF_ = 320000, 3
ATOM_V, BOND_V = 1152, 384
ATOM_B = N_NODES_ * NAF_
BOND_B = N_EDGES_ * NBF_

CHA = 312
ATOM_TAIL = NW * CHA
CHB = 400
BOND_PW = N_EDGES_ // NW
BOND_NCH = BOND_PW // CHB


def _sc_gather(x_flat, e_flat, atom_table, bond_table):
  mesh = plsc.VectorSubcoreMesh(core_axis_name="c", subcore_axis_name="s")

  @functools.partial(
      pl.kernel,
      out_type=(
          jax.ShapeDtypeStruct((ATOM_B, D), jnp.float32),
          jax.ShapeDtypeStruct((BOND_B, D), jnp.float32),
      ),
      mesh=mesh,
      scratch_types=[
          pltpu.VMEM_SHARED((ATOM_V, D), jnp.float32),
          pltpu.VMEM_SHARED((BOND_V, D), jnp.float32),
          pltpu.VMEM((CHA,), jnp.int32),
          pltpu.VMEM((CHA,), jnp.int32),
          pltpu.VMEM((BOND_PW,), jnp.int32),
          pltpu.VMEM((16,), jnp.int32),
          pltpu.VMEM((CHB, D), jnp.float32),
          pltpu.VMEM((CHB, D), jnp.float32),
          pltpu.VMEM((16, D), jnp.float32),
          pltpu.SemaphoreType.DMA,
          pltpu.SemaphoreType.DMA,
          pltpu.SemaphoreType.DMA,
          pltpu.SemaphoreType.DMA,
      ],
  )
  def k(x_hbm, e_hbm, at_hbm, bt_hbm, aout_hbm, bout_hbm,
        at_sp, bt_sp, aiA, aiB, bidx, ti, rA, rB, tr,
        gsA, gsB, ssA, ssB):
    sid = lax.axis_index("s")
    wid = sid * NC + lax.axis_index("c")

    @pl.when(sid == 0)
    def _():
      pltpu.sync_copy(at_hbm, at_sp)
      pltpu.sync_copy(bt_hbm, bt_sp)

    plsc.subcore_barrier()

    def run_pipe(tab_sp, out_hbm, row0_fn, ch, nch, idx_for, load_idx,
                 bufA, bufB):
      def start_gather(rbuf, sem, c, slot):
        load_idx(c, slot)
        return pltpu.async_copy(tab_sp.at[idx_for(c, slot)], rbuf, sem)

      def start_scatter(rbuf, sem, c):
        return pltpu.async_copy(
            rbuf, out_hbm.at[pl.ds(row0_fn(c), ch)], sem)

      def drain_scatter(rbuf, sem, c):
        pltpu.make_async_copy(
            rbuf, out_hbm.at[pl.ds(row0_fn(c), ch)], sem).wait()

      def pair(t, carry):
        c0 = 2 * t
        c1 = c0 + 1
        @pl.when(t > 0)
        def _():
          drain_scatter(bufA, ssA, c0)
          drain_scatter(bufB, ssB, c1)

        gA = start_gather(bufA, gsA, c0, 0)
        gB = start_gather(bufB, gsB, c1, 1)
        gA.wait()
        start_scatter(bufA, ssA, c0)
        gB.wait()
        start_scatter(bufB, ssB, c1)
        return carry

      lax.fori_loop(0, nch // 2, pair, 0)
      drain_scatter(bufA, ssA, 0)
      drain_scatter(bufB, ssB, 0)
      if nch % 2:
        c = nch - 1
        g = start_gather(bufA, gsA, c, 0)
        g.wait()
        start_scatter(bufA, ssA, c)
        drain_scatter(bufA, ssA, c)

    arA = rA.at[pl.ds(0, CHA)]
    arB = rB.at[pl.ds(0, CHA)]
    a_row0 = lambda c: c * N_NODES_ + wid * CHA

    def a_load_idx(c, slot):
      pltpu.sync_copy(x_hbm.at[pl.ds(a_row0(c), CHA)], (aiA, aiB)[slot])

    run_pipe(at_sp, aout_hbm, a_row0, CHA, NAF_,
             lambda c, slot: (aiA, aiB)[slot], a_load_idx, arA, arB)

    for f in range(NAF_):
      @pl.when(wid == f)
      def _(f=f):
        base = f * N_NODES_ + ATOM_TAIL
        pltpu.sync_copy(x_hbm.at[pl.ds(base, 16)], ti)
        pltpu.async_copy(at_sp.at[ti], tr, gsA).wait()
        pltpu.async_copy(tr, aout_hbm.at[pl.ds(base, 16)], ssA).wait()

    for f in range(NBF_):
      pbase = f * N_EDGES_ + wid * BOND_PW
      pltpu.sync_copy(e_hbm.at[pl.ds(pbase, BOND_PW)], bidx)
      run_pipe(bt_sp, bout_hbm, lambda c, pbase=pbase: pbase + c * CHB,
               CHB, BOND_NCH,
               lambda c, slot: bidx.at[pl.ds(c * CHB, CHB)],
               lambda c, slot: None, rA, rB)

  return k(x_flat, e_flat, atom_table, bond_table)


def kernel(x, edge_attr, atom_table, bond_table):
  x_flat = x.T.reshape(ATOM_B)
  e_flat = edge_attr.T.reshape(BOND_B)
  a_out, b_out = _sc_gather(x_flat, e_flat, atom_table, bond_table)
  return (
      a_out.reshape(NAF_, N_NODES_, D).transpose(1, 0, 2),
      b_out.reshape(NBF_, N_EDGES_, D).transpose(1, 0, 2),
  )
